# baseline (speedup 1.0000x reference)
import jax
import jax.numpy as jnp
from jax.experimental import pallas as pl
from jax.experimental.pallas import tpu as pltpu

_BS = 1024


def _add_kernel(xa_ref, xb_ref, t_ref, o_ref):
    j = pl.program_id(1)

    @pl.when(j % 2 == 0)
    def _():
        o_ref[...] = xa_ref[...] + t_ref[...]

    @pl.when(j % 2 == 1)
    def _():
        o_ref[...] = xb_ref[...] + t_ref[...]


def kernel(x, position_embeddings):
    B, S, D = x.shape
    table = position_embeddings[:S]
    grid = (S // _BS, B)
    return pl.pallas_call(
        _add_kernel,
        grid=grid,
        in_specs=[
            pl.BlockSpec((1, _BS, D), lambda i, j: (j // 2, i, 0)),
            pl.BlockSpec((1, _BS, D), lambda i, j: (2 + j // 2, i, 0)),
            pl.BlockSpec((_BS, D), lambda i, j: (i, 0)),
        ],
        out_specs=pl.BlockSpec((1, _BS, D),
                               lambda i, j: ((j % 2) * 2 + j // 2, i, 0)),
        out_shape=jax.ShapeDtypeStruct(x.shape, x.dtype),
    )(x, x, table)


# speedup vs baseline: 1.2717x; 1.2717x over previous
import jax
import jax.numpy as jnp
from jax import lax
from jax.experimental import pallas as pl
from jax.experimental.pallas import tpu as pltpu

_BS = 2048
_NBUF = 3


def _add_kernel(x_hbm, t_ref, o_ref, xs_ref, sems):
    i = pl.program_id(0)
    j = pl.program_id(1)
    nj = pl.num_programs(1)
    nk = pl.num_programs(0) * nj
    k = i * nj + j

    def _issue(kk):
        slot = lax.rem(kk, _NBUF)
        b = lax.rem(kk, nj)
        si = kk // nj
        pltpu.make_async_copy(
            x_hbm.at[b, pl.ds(si * _BS, _BS), :],
            xs_ref.at[slot],
            sems.at[slot],
        ).start()

    @pl.when(k == 0)
    def _():
        for kk in range(_NBUF):
            _issue(kk)

    slot = lax.rem(k, _NBUF)
    pltpu.make_async_copy(
        x_hbm.at[lax.rem(k, nj), pl.ds((k // nj) * _BS, _BS), :],
        xs_ref.at[slot],
        sems.at[slot],
    ).wait()

    o_ref[0] = xs_ref[slot] + t_ref[...]

    @pl.when(k + _NBUF < nk)
    def _():
        _issue(k + _NBUF)


def kernel(x, position_embeddings):
    B, S, D = x.shape
    table = position_embeddings[:S]
    grid = (S // _BS, B)
    return pl.pallas_call(
        _add_kernel,
        grid=grid,
        in_specs=[
            pl.BlockSpec(memory_space=pltpu.MemorySpace.HBM),
            pl.BlockSpec((_BS, D), lambda i, j: (i, 0)),
        ],
        out_specs=pl.BlockSpec((1, _BS, D), lambda i, j: (j, i, 0)),
        out_shape=jax.ShapeDtypeStruct(x.shape, x.dtype),
        scratch_shapes=[
            pltpu.VMEM((_NBUF, _BS, D), jnp.float32),
            pltpu.SemaphoreType.DMA((_NBUF,)),
        ],
    )(x, table)
